# baseline (speedup 1.0000x reference)
import functools

import jax
import jax.numpy as jnp
from jax import lax
from jax.experimental import pallas as pl
from jax.experimental.pallas import tpu as pltpu
from jax.experimental.pallas import tpu_sc as plsc

_VOCAB = 100000
_DIM = 64
_BATCH = 1024
_MAX_NORM = 1.0

_NUM_CORES = 2
_NUM_SUBCORES = 16
_NW = _NUM_CORES * _NUM_SUBCORES
_BPW = _BATCH // _NW

_V_BLK = 4096
_GRID = (_VOCAB + _V_BLK - 1) // _V_BLK
_TAIL = _VOCAB - (_GRID - 1) * _V_BLK
_NBUF = 2
_HALF = _V_BLK // 2
_HTAIL = _TAIL // 2
_TAIL_SLOT = (_GRID - 1) % _NBUF

_sc_gather_fn = None


def _get_sc_gather():
    global _sc_gather_fn
    if _sc_gather_fn is None:
        mesh = plsc.VectorSubcoreMesh(core_axis_name="c", subcore_axis_name="s")

        @functools.partial(
            pl.kernel,
            mesh=mesh,
            compiler_params=pltpu.CompilerParams(use_tc_tiling_on_sc=False),
            out_type=jax.ShapeDtypeStruct((_BATCH, _DIM), jnp.float32),
            scratch_types=[
                pltpu.VMEM((_BPW,), jnp.int32),
                pltpu.VMEM((_BPW, _DIM), jnp.float32),
                pltpu.SemaphoreType.DMA,
            ],
        )
        def sc_gather(table_hbm, idx_hbm, out_hbm, idx_v, rows_v, sem):
            wid = lax.axis_index("s") * _NUM_CORES + lax.axis_index("c")
            base = wid * _BPW
            pltpu.sync_copy(idx_hbm.at[pl.ds(base, _BPW)], idx_v)
            pltpu.async_copy(table_hbm.at[idx_v], rows_v, sem).wait()
            pltpu.sync_copy(rows_v, out_hbm.at[pl.ds(base, _BPW)])

        _sc_gather_fn = sc_gather
    return _sc_gather_fn


def _proj_body(emb_ref, w_ref, b_ref, out_hbm, x_ref, obufs, *sems):
    i = pl.program_id(0)

    @pl.when(i == 0)
    def _():
        emb = emb_ref[...]
        norm = jnp.sqrt(jnp.sum(emb * emb, axis=1, keepdims=True))
        scale = jnp.minimum(1.0, _MAX_NORM / jnp.maximum(norm, 1e-7))
        x_ref[...] = emb * scale

    result = lax.dot_general(
        w_ref[...], x_ref[...],
        (((0,), (1,)), ((), ())),
        preferred_element_type=jnp.float32,
    ) + b_ref[...]

    slot = lax.rem(i, _NBUF)
    for s in range(_NBUF):
        @pl.when(slot == s)
        def _(s=s):
            @pl.when(i >= _NBUF)
            def _():
                pltpu.make_async_copy(
                    obufs.at[s], out_hbm.at[pl.ds(0, _V_BLK)], sems[s]
                ).wait()
            obufs[s, ...] = result

            @pl.when(i < _GRID - 1)
            def _():
                for c in range(2):
                    pltpu.make_async_copy(
                        obufs.at[s, pl.ds(c * _HALF, _HALF)],
                        out_hbm.at[pl.ds(i * _V_BLK + c * _HALF, _HALF)],
                        sems[s],
                    ).start(priority=c)

            @pl.when(i == _GRID - 1)
            def _():
                for c in range(2):
                    pltpu.make_async_copy(
                        obufs.at[s, pl.ds(c * _HTAIL, _HTAIL)],
                        out_hbm.at[pl.ds(i * _V_BLK + c * _HTAIL, _HTAIL)],
                        sems[s],
                    ).start(priority=c)

    @pl.when(i == _GRID - 1)
    def _():
        for s in range(_NBUF):
            n_rows = _TAIL if s == _TAIL_SLOT else _V_BLK
            pltpu.make_async_copy(
                obufs.at[s, pl.ds(0, n_rows)], out_hbm.at[pl.ds(0, n_rows)], sems[s]
            ).wait()


def _projection_t(emb, W_t, b_col):
    return pl.pallas_call(
        _proj_body,
        grid=(_GRID,),
        in_specs=[
            pl.BlockSpec((_BATCH, _DIM), lambda i: (0, 0)),
            pl.BlockSpec((_DIM, _V_BLK), lambda i: (0, i)),
            pl.BlockSpec((_V_BLK, 1), lambda i: (i, 0)),
        ],
        out_specs=pl.BlockSpec(memory_space=pl.ANY),
        out_shape=jax.ShapeDtypeStruct((_VOCAB, _BATCH), jnp.float32),
        scratch_shapes=[
            pltpu.VMEM((_BATCH, _DIM), jnp.float32),
            pltpu.VMEM((_NBUF, _V_BLK, _BATCH), jnp.float32),
        ] + [pltpu.SemaphoreType.DMA] * _NBUF,
    )(emb, W_t, b_col)


def kernel(inputs_, table, W, b):
    emb = _get_sc_gather()(table, inputs_)
    out_t = _projection_t(emb, W.T, b.reshape(_VOCAB, 1))
    return out_t.T


# speedup vs baseline: 1.0031x; 1.0031x over previous
import functools

import jax
import jax.numpy as jnp
from jax import lax
from jax.experimental import pallas as pl
from jax.experimental.pallas import tpu as pltpu
from jax.experimental.pallas import tpu_sc as plsc

_VOCAB = 100000
_DIM = 64
_BATCH = 1024
_MAX_NORM = 1.0

_NUM_CORES = 2
_NUM_SUBCORES = 16
_NW = _NUM_CORES * _NUM_SUBCORES
_BPW = _BATCH // _NW

_V_BLK = 4096
_GRID = (_VOCAB + _V_BLK - 1) // _V_BLK
_TAIL = _VOCAB - (_GRID - 1) * _V_BLK
_NBUF = 2
_NCHUNK = 8
_CROWS = _V_BLK // _NCHUNK
_NCHUNK_TAIL = 4
_CTAIL = _TAIL // _NCHUNK_TAIL
_TAIL_SLOT = (_GRID - 1) % _NBUF

_sc_gather_fn = None


def _get_sc_gather():
    global _sc_gather_fn
    if _sc_gather_fn is None:
        mesh = plsc.VectorSubcoreMesh(core_axis_name="c", subcore_axis_name="s")

        @functools.partial(
            pl.kernel,
            mesh=mesh,
            compiler_params=pltpu.CompilerParams(use_tc_tiling_on_sc=False),
            out_type=jax.ShapeDtypeStruct((_BATCH, _DIM), jnp.float32),
            scratch_types=[
                pltpu.VMEM((_BPW,), jnp.int32),
                pltpu.VMEM((_BPW, _DIM), jnp.float32),
                pltpu.SemaphoreType.DMA,
            ],
        )
        def sc_gather(table_hbm, idx_hbm, out_hbm, idx_v, rows_v, sem):
            wid = lax.axis_index("s") * _NUM_CORES + lax.axis_index("c")
            base = wid * _BPW
            pltpu.sync_copy(idx_hbm.at[pl.ds(base, _BPW)], idx_v)
            pltpu.async_copy(table_hbm.at[idx_v], rows_v, sem).wait()
            pltpu.sync_copy(rows_v, out_hbm.at[pl.ds(base, _BPW)])

        _sc_gather_fn = sc_gather
    return _sc_gather_fn


def _proj_body(emb_ref, w_ref, b_ref, out_hbm, x_ref, obufs, *sems):
    i = pl.program_id(0)

    @pl.when(i == 0)
    def _():
        emb = emb_ref[...]
        norm = jnp.sqrt(jnp.sum(emb * emb, axis=1, keepdims=True))
        scale = jnp.minimum(1.0, _MAX_NORM / jnp.maximum(norm, 1e-7))
        x_ref[...] = emb * scale

    result = lax.dot_general(
        w_ref[...], x_ref[...],
        (((0,), (1,)), ((), ())),
        preferred_element_type=jnp.float32,
    ) + b_ref[...]

    slot = lax.rem(i, _NBUF)
    for s in range(_NBUF):
        @pl.when(slot == s)
        def _(s=s):
            @pl.when(i >= _NBUF)
            def _():
                pltpu.make_async_copy(
                    obufs.at[s], out_hbm.at[pl.ds(0, _V_BLK)], sems[s]
                ).wait()
            obufs[s, ...] = result

            @pl.when(i < _GRID - 1)
            def _():
                for c in range(_NCHUNK):
                    pltpu.make_async_copy(
                        obufs.at[s, pl.ds(c * _CROWS, _CROWS)],
                        out_hbm.at[pl.ds(i * _V_BLK + c * _CROWS, _CROWS)],
                        sems[s],
                    ).start(priority=c % 2)

            @pl.when(i == _GRID - 1)
            def _():
                for c in range(_NCHUNK_TAIL):
                    pltpu.make_async_copy(
                        obufs.at[s, pl.ds(c * _CTAIL, _CTAIL)],
                        out_hbm.at[pl.ds(i * _V_BLK + c * _CTAIL, _CTAIL)],
                        sems[s],
                    ).start(priority=c % 2)

    @pl.when(i == _GRID - 1)
    def _():
        for s in range(_NBUF):
            n_rows = _TAIL if s == _TAIL_SLOT else _V_BLK
            pltpu.make_async_copy(
                obufs.at[s, pl.ds(0, n_rows)], out_hbm.at[pl.ds(0, n_rows)], sems[s]
            ).wait()


def _projection_t(emb, W_t, b_col):
    return pl.pallas_call(
        _proj_body,
        grid=(_GRID,),
        in_specs=[
            pl.BlockSpec((_BATCH, _DIM), lambda i: (0, 0)),
            pl.BlockSpec((_DIM, _V_BLK), lambda i: (0, i)),
            pl.BlockSpec((_V_BLK, 1), lambda i: (i, 0)),
        ],
        out_specs=pl.BlockSpec(memory_space=pl.ANY),
        out_shape=jax.ShapeDtypeStruct((_VOCAB, _BATCH), jnp.float32),
        scratch_shapes=[
            pltpu.VMEM((_BATCH, _DIM), jnp.float32),
            pltpu.VMEM((_NBUF, _V_BLK, _BATCH), jnp.float32),
        ] + [pltpu.SemaphoreType.DMA] * _NBUF,
    )(emb, W_t, b_col)


def kernel(inputs_, table, W, b):
    emb = _get_sc_gather()(table, inputs_)
    out_t = _projection_t(emb, W.T, b.reshape(_VOCAB, 1))
    return out_t.T


# speedup vs baseline: 1.0190x; 1.0159x over previous
import functools

import jax
import jax.numpy as jnp
from jax import lax
from jax.experimental import pallas as pl
from jax.experimental.pallas import tpu as pltpu
from jax.experimental.pallas import tpu_sc as plsc

_VOCAB = 100000
_DIM = 64
_BATCH = 1024
_MAX_NORM = 1.0

_NUM_CORES = 2
_NUM_SUBCORES = 16
_NW = _NUM_CORES * _NUM_SUBCORES
_BPW = _BATCH // _NW

_V_BLK = 4096
_GRID = (_VOCAB + _V_BLK - 1) // _V_BLK

_sc_gather_fn = None


def _get_sc_gather():
    global _sc_gather_fn
    if _sc_gather_fn is None:
        mesh = plsc.VectorSubcoreMesh(core_axis_name="c", subcore_axis_name="s")

        @functools.partial(
            pl.kernel,
            mesh=mesh,
            compiler_params=pltpu.CompilerParams(use_tc_tiling_on_sc=False),
            out_type=jax.ShapeDtypeStruct((_BATCH, _DIM), jnp.float32),
            scratch_types=[
                pltpu.VMEM((_BPW,), jnp.int32),
                pltpu.VMEM((_BPW, _DIM), jnp.float32),
                pltpu.SemaphoreType.DMA,
            ],
        )
        def sc_gather(table_hbm, idx_hbm, out_hbm, idx_v, rows_v, sem):
            wid = lax.axis_index("s") * _NUM_CORES + lax.axis_index("c")
            base = wid * _BPW
            pltpu.sync_copy(idx_hbm.at[pl.ds(base, _BPW)], idx_v)
            pltpu.async_copy(table_hbm.at[idx_v], rows_v, sem).wait()
            pltpu.sync_copy(rows_v, out_hbm.at[pl.ds(base, _BPW)])

        _sc_gather_fn = sc_gather
    return _sc_gather_fn


def _proj_body(emb_ref, w_ref, b_ref, out_ref, x_ref):
    @pl.when(pl.program_id(0) == 0)
    def _():
        emb = emb_ref[...]
        norm = jnp.sqrt(jnp.sum(emb * emb, axis=1, keepdims=True))
        scale = jnp.minimum(1.0, _MAX_NORM / jnp.maximum(norm, 1e-7))
        x_ref[...] = emb * scale

    out_ref[...] = lax.dot_general(
        w_ref[...], x_ref[...],
        (((0,), (1,)), ((), ())),
        preferred_element_type=jnp.float32,
    ) + b_ref[...]


def _projection_t(emb, W_t, b_col):
    return pl.pallas_call(
        _proj_body,
        grid=(_GRID,),
        in_specs=[
            pl.BlockSpec((_BATCH, _DIM), lambda i: (0, 0)),
            pl.BlockSpec((_DIM, _V_BLK), lambda i: (0, i)),
            pl.BlockSpec((_V_BLK, 1), lambda i: (i, 0)),
        ],
        out_specs=pl.BlockSpec((_V_BLK, _BATCH), lambda i: (i, 0)),
        out_shape=jax.ShapeDtypeStruct((_VOCAB, _BATCH), jnp.float32),
        scratch_shapes=[pltpu.VMEM((_BATCH, _DIM), jnp.float32)],
    )(emb, W_t, b_col)


def kernel(inputs_, table, W, b):
    emb = _get_sc_gather()(table, inputs_)
    out_t = _projection_t(emb, W.T, b.reshape(_VOCAB, 1))
    return out_t.T
